# baseline (speedup 1.0000x reference)
import jax
import jax.numpy as jnp
from jax import lax
from jax.experimental import pallas as pl
from jax.experimental.pallas import tpu as pltpu
from jax.experimental.pallas import tpu_sc as plsc

_N = 10000
_E = 160000
_DIM = 64
_B = 512


def _leaky(v):
    return jnp.where(v >= 0, v, 0.01 * v)


def _lin0_body(x_ref, w_ref, b_ref, o_ref):
    o_ref[...] = _leaky(
        jnp.dot(x_ref[...], w_ref[...], preferred_element_type=jnp.float32)
        + b_ref[...])


def _lin0(x, w_t, b2):
    n = x.shape[0]
    return pl.pallas_call(
        _lin0_body,
        out_shape=jax.ShapeDtypeStruct((n, _DIM), jnp.float32),
    )(x, w_t, b2)


def _hidt_body(eat_ref, w_ref, b_ref, o_ref):
    o_ref[...] = _leaky(
        jnp.dot(w_ref[...], eat_ref[...], preferred_element_type=jnp.float32)
        + b_ref[...]).astype(jnp.bfloat16)


def _hidt(ea_t, net1_W, net1_bc):
    e = ea_t.shape[1]
    te = 6400
    return pl.pallas_call(
        _hidt_body,
        grid=(e // te,),
        in_specs=[
            pl.BlockSpec((4, te), lambda i: (0, i)),
            pl.BlockSpec((128, 4), lambda i: (0, 0)),
            pl.BlockSpec((128, 1), lambda i: (0, 0)),
        ],
        out_specs=pl.BlockSpec((128, te), lambda i: (0, i)),
        out_shape=jax.ShapeDtypeStruct((128, e), jnp.bfloat16),
    )(ea_t, net1_W, net1_bc)


def _msg_body(hidt_ref, sf_ref, w2_ref, b2t_ref, o_ref):
    w_t = jnp.dot(w2_ref[...], hidt_ref[...],
                  preferred_element_type=jnp.float32)
    sft = sf_ref[...].T
    acc = jnp.dot(b2t_ref[...], sft,
                  preferred_element_type=jnp.float32)
    for i in range(_DIM):
        acc = acc + sft[i:i + 1, :] * w_t[i * _DIM:(i + 1) * _DIM, :]
    o_ref[...] = acc.T


def _msg(hid_t, sf, net2_W, b2t, te=256):
    e = sf.shape[0]
    return pl.pallas_call(
        _msg_body,
        grid=(e // te,),
        in_specs=[
            pl.BlockSpec((128, te), lambda i: (0, i)),
            pl.BlockSpec((te, _DIM), lambda i: (i, 0)),
            pl.BlockSpec((4096, 128), lambda i: (0, 0)),
            pl.BlockSpec((_DIM, _DIM), lambda i: (0, 0)),
        ],
        out_specs=pl.BlockSpec((te, _DIM), lambda i: (i, 0)),
        out_shape=jax.ShapeDtypeStruct((e, _DIM), jnp.float32),
    )(hid_t, sf, net2_W, b2t)


def _dense_body(st_ref, a0_ref, a1_ref, a2_ref, a3_ref, d0_ref, d1_ref,
                rw_ref, cb_ref, wih_ref, whh_ref, bih_ref, bhh_ref, o_ref):
    d = d0_ref[...] + d1_ref[...]
    invd = 1.0 / jnp.maximum(d, 1.0)
    aggr = (a0_ref[...] + a1_ref[...] + a2_ref[...] + a3_ref[...]) * invd
    st = st_ref[...]
    m = _leaky(jnp.dot(st, rw_ref[...], preferred_element_type=jnp.float32)
               + aggr + cb_ref[...])
    gi = jnp.dot(m, wih_ref[...], preferred_element_type=jnp.float32) + bih_ref[...]
    gh = jnp.dot(st, whh_ref[...], preferred_element_type=jnp.float32) + bhh_ref[...]
    r = jax.nn.sigmoid(gi[:, 0:64] + gh[:, 0:64])
    z = jax.nn.sigmoid(gi[:, 64:128] + gh[:, 64:128])
    n = jnp.tanh(gi[:, 128:192] + r * gh[:, 128:192])
    o_ref[...] = (1.0 - z) * n + z * st


def _dense(st, ag2a, ag2b, dg2, root_W, conv_b2, wih_t, whh_t, bih2, bhh2,
           nb=2000):
    n = st.shape[0]
    nblk = n // nb
    return pl.pallas_call(
        _dense_body,
        grid=(nblk,),
        in_specs=[
            pl.BlockSpec((nb, _DIM), lambda i: (i, 0)),
            pl.BlockSpec((nb, _DIM), lambda i: (i, 0)),
            pl.BlockSpec((nb, _DIM), lambda i, _n=nblk: (i + _n, 0)),
            pl.BlockSpec((nb, _DIM), lambda i: (i, 0)),
            pl.BlockSpec((nb, _DIM), lambda i, _n=nblk: (i + _n, 0)),
            pl.BlockSpec((nb, 1), lambda i: (i, 0)),
            pl.BlockSpec((nb, 1), lambda i, _n=nblk: (i + _n, 0)),
            pl.BlockSpec((_DIM, _DIM), lambda i: (0, 0)),
            pl.BlockSpec((1, _DIM), lambda i: (0, 0)),
            pl.BlockSpec((_DIM, 192), lambda i: (0, 0)),
            pl.BlockSpec((_DIM, 192), lambda i: (0, 0)),
            pl.BlockSpec((1, 192), lambda i: (0, 0)),
            pl.BlockSpec((1, 192), lambda i: (0, 0)),
        ],
        out_specs=pl.BlockSpec((nb, _DIM), lambda i: (i, 0)),
        out_shape=jax.ShapeDtypeStruct((n, _DIM), jnp.float32),
    )(st, ag2a, ag2a, ag2b, ag2b, dg2, dg2, root_W, conv_b2, wih_t, whh_t,
      bih2, bhh2)


def _s2s_body(out_ref, bat_ref, batt_ref, wih_ref, whh_ref, bb_ref,
              l3w_ref, l3b_ref, o_ref, e_scr):
    nblk = bat_ref.shape[0]
    nb = bat_ref.shape[1]
    grow = lax.broadcasted_iota(jnp.int32, (1, _B), 1)
    gcol = lax.broadcasted_iota(jnp.int32, (_B, 1), 0)

    q_star = jnp.zeros((_B, 2 * _DIM), jnp.float32)
    hx = jnp.zeros((_B, _DIM), jnp.float32)
    cx = jnp.zeros((_B, _DIM), jnp.float32)
    for _ in range(3):
        g = (jnp.dot(q_star, wih_ref[...], preferred_element_type=jnp.float32)
             + jnp.dot(hx, whh_ref[...], preferred_element_type=jnp.float32)
             + bb_ref[...])
        i_g = jax.nn.sigmoid(g[:, 0:64])
        f_g = jax.nn.sigmoid(g[:, 64:128])
        c_g = jnp.tanh(g[:, 128:192])
        o_g = jax.nn.sigmoid(g[:, 192:256])
        cx = f_g * cx + i_g * c_g
        hx = o_g * jnp.tanh(cx)
        q = hx

        emax_row = jnp.full((1, _B), -jnp.inf, jnp.float32)
        for b in range(nblk):
            out_blk = out_ref[b * nb:(b + 1) * nb, :]
            ids_col = batt_ref[:, b:b + 1]
            mask = ids_col == grow
            q_g = jnp.dot(mask.astype(jnp.float32), q,
                          preferred_element_type=jnp.float32)
            e_col = jnp.sum(out_blk * q_g, axis=1, keepdims=True)
            e_scr[:, b:b + 1] = e_col
            contrib = jnp.where(mask, e_col, -jnp.inf)
            emax_row = jnp.maximum(
                emax_row, jnp.max(contrib, axis=0, keepdims=True))
        emax_row = jnp.where(emax_row > -1e37, emax_row, 0.0)

        asum = jnp.zeros((_B, 1), jnp.float32)
        rread = jnp.zeros((_B, _DIM), jnp.float32)
        for b in range(nblk):
            out_blk = out_ref[b * nb:(b + 1) * nb, :]
            ids_col = batt_ref[:, b:b + 1]
            ids_row = bat_ref[b:b + 1, :]
            mask_f = (ids_col == grow).astype(jnp.float32)
            maskt_f = (gcol == ids_row).astype(jnp.float32)
            e_col = e_scr[:, b:b + 1]
            emax_g = jnp.sum(mask_f * emax_row, axis=1, keepdims=True)
            a_col = jnp.exp(e_col - emax_g)
            asum = asum + jnp.dot(maskt_f, a_col,
                                  preferred_element_type=jnp.float32)
            rread = rread + jnp.dot(maskt_f, a_col * out_blk,
                                    preferred_element_type=jnp.float32)
        rread = rread / jnp.maximum(asum, 1e-16)
        q_star = jnp.concatenate([q, rread], axis=1)

    o_ref[...] = (jnp.dot(q_star, l3w_ref[...],
                          preferred_element_type=jnp.float32) + l3b_ref[...])


def _s2s(st, bat2, bat2t, wih_t, whh_t, bb2, l3w_t, l3b2):
    return pl.pallas_call(
        _s2s_body,
        out_shape=jax.ShapeDtypeStruct((_B, 1), jnp.float32),
        scratch_shapes=[pltpu.VMEM((bat2t.shape[0], bat2.shape[0]),
                                   jnp.float32)],
    )(st, bat2, bat2t, wih_t, whh_t, bb2, l3w_t, l3b2)


def _pick_ch(e_per_w):
    best = 8
    for d in range(8, 1001, 8):
        if e_per_w % d == 0:
            best = d
    return best


def _gather_sc(table, idx):
    e = idx.shape[0]
    e_per_w = e // 32
    ch = _pick_ch(e_per_w)
    n_ch = e_per_w // ch
    mesh = plsc.VectorSubcoreMesh(core_axis_name="c", subcore_axis_name="s")

    def body(table_ref, idx_ref, out_ref, idx_v, rows_v, sem):
        wid = lax.axis_index("c") * 16 + lax.axis_index("s")
        base = wid * e_per_w

        def chunk(j, carry):
            off = base + j * ch
            pltpu.sync_copy(idx_ref.at[pl.ds(off, ch)], idx_v)
            pltpu.async_copy(table_ref.at[idx_v], rows_v, sem).wait()
            pltpu.sync_copy(rows_v, out_ref.at[pl.ds(off, ch)])
            return carry

        lax.fori_loop(0, n_ch, chunk, 0)

    f = pl.kernel(
        body,
        out_type=jax.ShapeDtypeStruct((e, _DIM), jnp.float32),
        mesh=mesh,
        scratch_types=[
            pltpu.VMEM((ch,), jnp.int32),
            pltpu.VMEM((ch, _DIM), jnp.float32),
            pltpu.SemaphoreType.DMA,
        ],
        compiler_params=pltpu.CompilerParams(use_tc_tiling_on_sc=False),
    )
    return f(table, idx)


def _scatter_sc(msg, dst, zrows):
    e = msg.shape[0]
    n = _N
    rpt = n // 16
    e_per_w = e // 32
    ch = _pick_ch(e_per_w)
    n_ch = e_per_w // ch
    mesh = plsc.VectorSubcoreMesh(core_axis_name="c", subcore_axis_name="s")

    def body(msg_ref, dst_ref, z_ref, out_ref, idx_v, buf_v, acc_sh):
        cid = lax.axis_index("c")
        sid = lax.axis_index("s")
        pltpu.sync_copy(z_ref, acc_sh.at[pl.ds(sid * rpt, rpt)])
        plsc.subcore_barrier()
        base = cid * (16 * e_per_w) + sid * e_per_w

        def chunk(j, carry):
            off = base + j * ch
            pltpu.sync_copy(dst_ref.at[pl.ds(off, ch)], idx_v)
            pltpu.sync_copy(msg_ref.at[pl.ds(off, ch)], buf_v)
            pltpu.sync_copy(buf_v, acc_sh.at[idx_v], add=True)
            return carry

        lax.fori_loop(0, n_ch, chunk, 0)
        plsc.subcore_barrier()
        pltpu.sync_copy(acc_sh.at[pl.ds(sid * rpt, rpt)],
                        out_ref.at[pl.ds(cid * n + sid * rpt, rpt)])

    f = pl.kernel(
        body,
        out_type=jax.ShapeDtypeStruct((2 * n, _DIM), jnp.float32),
        mesh=mesh,
        scratch_types=[
            pltpu.VMEM((ch,), jnp.int32),
            pltpu.VMEM((ch, _DIM), jnp.float32),
            pltpu.VMEM_SHARED((n, _DIM), jnp.float32),
        ],
        compiler_params=pltpu.CompilerParams(use_tc_tiling_on_sc=False),
    )
    return f(msg, dst, zrows)


def kernel(x, edge_index, edge_attr, batch, lin0_W, lin0_b, net1_W, net1_b,
           net2_W, net2_b, root_W, conv_b, gru_W_ih, gru_W_hh, gru_b_ih,
           gru_b_hh, lstm_W_ih, lstm_W_hh, lstm_b_ih, lstm_b_hh, lin3_W,
           lin3_b):
    src = edge_index[0]
    dst = edge_index[1]

    st = _lin0(x, lin0_W.T, lin0_b[None, :])
    hid_t = _hidt(edge_attr.T, net1_W, net1_b[:, None])
    net2_Wb = net2_W.astype(jnp.bfloat16)
    b2t = net2_b.reshape(_DIM, _DIM).T

    h1 = 81920
    src1, src2 = src[:h1], src[h1:]
    dst1, dst2 = dst[:h1], dst[h1:]
    hid_t1, hid_t2 = hid_t[:, :h1], hid_t[:, h1:]

    zrows = jnp.zeros((_N // 16, _DIM), jnp.float32)
    ones_rows = jnp.ones((_E, _DIM), jnp.float32)
    dg2 = _scatter_sc(ones_rows, dst, zrows)[:, 0:1]

    wih_t = gru_W_ih.T
    whh_t = gru_W_hh.T
    bih2 = gru_b_ih[None, :]
    bhh2 = gru_b_hh[None, :]
    conv_b2 = conv_b[None, :]

    for _ in range(12):
        sf1 = _gather_sc(st, src1)
        sf2 = _gather_sc(st, src2)
        msg1 = _msg(hid_t1, sf1, net2_Wb, b2t)
        ag2a = _scatter_sc(msg1, dst1, zrows)
        msg2 = _msg(hid_t2, sf2, net2_Wb, b2t)
        ag2b = _scatter_sc(msg2, dst2, zrows)
        st = _dense(st, ag2a, ag2b, dg2, root_W, conv_b2, wih_t, whh_t,
                    bih2, bhh2)

    bat2 = batch.reshape(20, 500)
    bat2t = bat2.T
    out = _s2s(st, bat2, bat2t, lstm_W_ih.T, lstm_W_hh.T,
               (lstm_b_ih + lstm_b_hh)[None, :], lin3_W.T, lin3_b[None, :])
    return out


# speedup vs baseline: 1.0670x; 1.0670x over previous
import jax
import jax.numpy as jnp
from jax import lax
from jax.experimental import pallas as pl
from jax.experimental.pallas import tpu as pltpu
from jax.experimental.pallas import tpu_sc as plsc

_N = 10000
_E = 160000
_DIM = 64
_B = 512


def _leaky(v):
    return jnp.where(v >= 0, v, 0.01 * v)


def _lin0_body(x_ref, w_ref, b_ref, o_ref):
    o_ref[...] = _leaky(
        jnp.dot(x_ref[...], w_ref[...], preferred_element_type=jnp.float32)
        + b_ref[...])


def _lin0(x, w_t, b2):
    n = x.shape[0]
    return pl.pallas_call(
        _lin0_body,
        out_shape=jax.ShapeDtypeStruct((n, _DIM), jnp.float32),
    )(x, w_t, b2)


def _hidt_body(eat_ref, w_ref, b_ref, o_ref):
    o_ref[...] = _leaky(
        jnp.dot(w_ref[...], eat_ref[...], preferred_element_type=jnp.float32)
        + b_ref[...]).astype(jnp.bfloat16)


def _hidt(ea_t, net1_W, net1_bc):
    e = ea_t.shape[1]
    te = 6400
    return pl.pallas_call(
        _hidt_body,
        grid=(e // te,),
        in_specs=[
            pl.BlockSpec((4, te), lambda i: (0, i)),
            pl.BlockSpec((128, 4), lambda i: (0, 0)),
            pl.BlockSpec((128, 1), lambda i: (0, 0)),
        ],
        out_specs=pl.BlockSpec((128, te), lambda i: (0, i)),
        out_shape=jax.ShapeDtypeStruct((128, e), jnp.bfloat16),
    )(ea_t, net1_W, net1_bc)


def _msg_body(hidt_ref, sf_ref, w2_ref, b2t_ref, o_ref):
    hid = hidt_ref[...]
    sft = sf_ref[...].T
    acc = jnp.dot(b2t_ref[...], sft,
                  preferred_element_type=jnp.float32)
    for i in range(_DIM):
        part = jnp.dot(w2_ref[i * _DIM:(i + 1) * _DIM, :], hid,
                       preferred_element_type=jnp.float32)
        acc = acc + sft[i:i + 1, :] * part
    o_ref[...] = acc.T


def _msg(hid_t, sf, net2_W, b2t, te=256):
    e = sf.shape[0]
    return pl.pallas_call(
        _msg_body,
        grid=(e // te,),
        in_specs=[
            pl.BlockSpec((128, te), lambda i: (0, i)),
            pl.BlockSpec((te, _DIM), lambda i: (i, 0)),
            pl.BlockSpec((4096, 128), lambda i: (0, 0)),
            pl.BlockSpec((_DIM, _DIM), lambda i: (0, 0)),
        ],
        out_specs=pl.BlockSpec((te, _DIM), lambda i: (i, 0)),
        out_shape=jax.ShapeDtypeStruct((e, _DIM), jnp.float32),
    )(hid_t, sf, net2_W, b2t)


def _dense_body(st_ref, a0_ref, a1_ref, a2_ref, a3_ref, d0_ref, d1_ref,
                rw_ref, cb_ref, wih_ref, whh_ref, bih_ref, bhh_ref, o_ref):
    d = d0_ref[...] + d1_ref[...]
    invd = 1.0 / jnp.maximum(d, 1.0)
    aggr = (a0_ref[...] + a1_ref[...] + a2_ref[...] + a3_ref[...]) * invd
    st = st_ref[...]
    m = _leaky(jnp.dot(st, rw_ref[...], preferred_element_type=jnp.float32)
               + aggr + cb_ref[...])
    gi = jnp.dot(m, wih_ref[...], preferred_element_type=jnp.float32) + bih_ref[...]
    gh = jnp.dot(st, whh_ref[...], preferred_element_type=jnp.float32) + bhh_ref[...]
    r = jax.nn.sigmoid(gi[:, 0:64] + gh[:, 0:64])
    z = jax.nn.sigmoid(gi[:, 64:128] + gh[:, 64:128])
    n = jnp.tanh(gi[:, 128:192] + r * gh[:, 128:192])
    o_ref[...] = (1.0 - z) * n + z * st


def _dense(st, ag2a, ag2b, dg2, root_W, conv_b2, wih_t, whh_t, bih2, bhh2,
           nb=2000):
    n = st.shape[0]
    nblk = n // nb
    return pl.pallas_call(
        _dense_body,
        grid=(nblk,),
        in_specs=[
            pl.BlockSpec((nb, _DIM), lambda i: (i, 0)),
            pl.BlockSpec((nb, _DIM), lambda i: (i, 0)),
            pl.BlockSpec((nb, _DIM), lambda i, _n=nblk: (i + _n, 0)),
            pl.BlockSpec((nb, _DIM), lambda i: (i, 0)),
            pl.BlockSpec((nb, _DIM), lambda i, _n=nblk: (i + _n, 0)),
            pl.BlockSpec((nb, 1), lambda i: (i, 0)),
            pl.BlockSpec((nb, 1), lambda i, _n=nblk: (i + _n, 0)),
            pl.BlockSpec((_DIM, _DIM), lambda i: (0, 0)),
            pl.BlockSpec((1, _DIM), lambda i: (0, 0)),
            pl.BlockSpec((_DIM, 192), lambda i: (0, 0)),
            pl.BlockSpec((_DIM, 192), lambda i: (0, 0)),
            pl.BlockSpec((1, 192), lambda i: (0, 0)),
            pl.BlockSpec((1, 192), lambda i: (0, 0)),
        ],
        out_specs=pl.BlockSpec((nb, _DIM), lambda i: (i, 0)),
        out_shape=jax.ShapeDtypeStruct((n, _DIM), jnp.float32),
    )(st, ag2a, ag2a, ag2b, ag2b, dg2, dg2, root_W, conv_b2, wih_t, whh_t,
      bih2, bhh2)


def _s2s_body(out_ref, bat_ref, batt_ref, wih_ref, whh_ref, bb_ref,
              l3w_ref, l3b_ref, o_ref, e_scr):
    nblk = bat_ref.shape[0]
    nb = bat_ref.shape[1]
    grow = lax.broadcasted_iota(jnp.int32, (1, _B), 1)
    gcol = lax.broadcasted_iota(jnp.int32, (_B, 1), 0)

    q_star = jnp.zeros((_B, 2 * _DIM), jnp.float32)
    hx = jnp.zeros((_B, _DIM), jnp.float32)
    cx = jnp.zeros((_B, _DIM), jnp.float32)
    for _ in range(3):
        g = (jnp.dot(q_star, wih_ref[...], preferred_element_type=jnp.float32)
             + jnp.dot(hx, whh_ref[...], preferred_element_type=jnp.float32)
             + bb_ref[...])
        i_g = jax.nn.sigmoid(g[:, 0:64])
        f_g = jax.nn.sigmoid(g[:, 64:128])
        c_g = jnp.tanh(g[:, 128:192])
        o_g = jax.nn.sigmoid(g[:, 192:256])
        cx = f_g * cx + i_g * c_g
        hx = o_g * jnp.tanh(cx)
        q = hx

        emax_row = jnp.full((1, _B), -jnp.inf, jnp.float32)
        for b in range(nblk):
            out_blk = out_ref[b * nb:(b + 1) * nb, :]
            ids_col = batt_ref[:, b:b + 1]
            mask = ids_col == grow
            q_g = jnp.dot(mask.astype(jnp.float32), q,
                          preferred_element_type=jnp.float32)
            e_col = jnp.sum(out_blk * q_g, axis=1, keepdims=True)
            e_scr[:, b:b + 1] = e_col
            contrib = jnp.where(mask, e_col, -jnp.inf)
            emax_row = jnp.maximum(
                emax_row, jnp.max(contrib, axis=0, keepdims=True))
        emax_row = jnp.where(emax_row > -1e37, emax_row, 0.0)

        asum = jnp.zeros((_B, 1), jnp.float32)
        rread = jnp.zeros((_B, _DIM), jnp.float32)
        for b in range(nblk):
            out_blk = out_ref[b * nb:(b + 1) * nb, :]
            ids_col = batt_ref[:, b:b + 1]
            ids_row = bat_ref[b:b + 1, :]
            mask_f = (ids_col == grow).astype(jnp.float32)
            maskt_f = (gcol == ids_row).astype(jnp.float32)
            e_col = e_scr[:, b:b + 1]
            emax_g = jnp.sum(mask_f * emax_row, axis=1, keepdims=True)
            a_col = jnp.exp(e_col - emax_g)
            asum = asum + jnp.dot(maskt_f, a_col,
                                  preferred_element_type=jnp.float32)
            rread = rread + jnp.dot(maskt_f, a_col * out_blk,
                                    preferred_element_type=jnp.float32)
        rread = rread / jnp.maximum(asum, 1e-16)
        q_star = jnp.concatenate([q, rread], axis=1)

    o_ref[...] = (jnp.dot(q_star, l3w_ref[...],
                          preferred_element_type=jnp.float32) + l3b_ref[...])


def _s2s(st, bat2, bat2t, wih_t, whh_t, bb2, l3w_t, l3b2):
    return pl.pallas_call(
        _s2s_body,
        out_shape=jax.ShapeDtypeStruct((_B, 1), jnp.float32),
        scratch_shapes=[pltpu.VMEM((bat2t.shape[0], bat2.shape[0]),
                                   jnp.float32)],
    )(st, bat2, bat2t, wih_t, whh_t, bb2, l3w_t, l3b2)


def _pick_ch(e_per_w):
    best = 8
    for d in range(8, 1001, 8):
        if e_per_w % d == 0:
            best = d
    return best


def _gather_sc(table, idx):
    e = idx.shape[0]
    e_per_w = e // 32
    ch = _pick_ch(e_per_w)
    n_ch = e_per_w // ch
    mesh = plsc.VectorSubcoreMesh(core_axis_name="c", subcore_axis_name="s")

    def body(table_ref, idx_ref, out_ref, idx_v, rows_v, sem):
        wid = lax.axis_index("c") * 16 + lax.axis_index("s")
        base = wid * e_per_w

        def chunk(j, carry):
            off = base + j * ch
            pltpu.sync_copy(idx_ref.at[pl.ds(off, ch)], idx_v)
            pltpu.async_copy(table_ref.at[idx_v], rows_v, sem).wait()
            pltpu.sync_copy(rows_v, out_ref.at[pl.ds(off, ch)])
            return carry

        lax.fori_loop(0, n_ch, chunk, 0)

    f = pl.kernel(
        body,
        out_type=jax.ShapeDtypeStruct((e, _DIM), jnp.float32),
        mesh=mesh,
        scratch_types=[
            pltpu.VMEM((ch,), jnp.int32),
            pltpu.VMEM((ch, _DIM), jnp.float32),
            pltpu.SemaphoreType.DMA,
        ],
        compiler_params=pltpu.CompilerParams(use_tc_tiling_on_sc=False),
    )
    return f(table, idx)


def _scatter_sc(msg, dst, zrows):
    e = msg.shape[0]
    n = _N
    rpt = n // 16
    e_per_w = e // 32
    ch = _pick_ch(e_per_w)
    n_ch = e_per_w // ch
    mesh = plsc.VectorSubcoreMesh(core_axis_name="c", subcore_axis_name="s")

    def body(msg_ref, dst_ref, z_ref, out_ref, idx_v, buf_v, acc_sh):
        cid = lax.axis_index("c")
        sid = lax.axis_index("s")
        pltpu.sync_copy(z_ref, acc_sh.at[pl.ds(sid * rpt, rpt)])
        plsc.subcore_barrier()
        base = cid * (16 * e_per_w) + sid * e_per_w

        def chunk(j, carry):
            off = base + j * ch
            pltpu.sync_copy(dst_ref.at[pl.ds(off, ch)], idx_v)
            pltpu.sync_copy(msg_ref.at[pl.ds(off, ch)], buf_v)
            pltpu.sync_copy(buf_v, acc_sh.at[idx_v], add=True)
            return carry

        lax.fori_loop(0, n_ch, chunk, 0)
        plsc.subcore_barrier()
        pltpu.sync_copy(acc_sh.at[pl.ds(sid * rpt, rpt)],
                        out_ref.at[pl.ds(cid * n + sid * rpt, rpt)])

    f = pl.kernel(
        body,
        out_type=jax.ShapeDtypeStruct((2 * n, _DIM), jnp.float32),
        mesh=mesh,
        scratch_types=[
            pltpu.VMEM((ch,), jnp.int32),
            pltpu.VMEM((ch, _DIM), jnp.float32),
            pltpu.VMEM_SHARED((n, _DIM), jnp.float32),
        ],
        compiler_params=pltpu.CompilerParams(use_tc_tiling_on_sc=False),
    )
    return f(msg, dst, zrows)


def kernel(x, edge_index, edge_attr, batch, lin0_W, lin0_b, net1_W, net1_b,
           net2_W, net2_b, root_W, conv_b, gru_W_ih, gru_W_hh, gru_b_ih,
           gru_b_hh, lstm_W_ih, lstm_W_hh, lstm_b_ih, lstm_b_hh, lin3_W,
           lin3_b):
    src = edge_index[0]
    dst = edge_index[1]

    st = _lin0(x, lin0_W.T, lin0_b[None, :])
    hid_t = _hidt(edge_attr.T, net1_W, net1_b[:, None])
    net2_Wb = net2_W.astype(jnp.bfloat16)
    b2t = net2_b.reshape(_DIM, _DIM).T

    h1 = 81920
    src1, src2 = src[:h1], src[h1:]
    dst1, dst2 = dst[:h1], dst[h1:]
    hid_t1, hid_t2 = hid_t[:, :h1], hid_t[:, h1:]

    zrows = jnp.zeros((_N // 16, _DIM), jnp.float32)
    ones_rows = jnp.ones((_E, _DIM), jnp.float32)
    dg2 = _scatter_sc(ones_rows, dst, zrows)[:, 0:1]

    wih_t = gru_W_ih.T
    whh_t = gru_W_hh.T
    bih2 = gru_b_ih[None, :]
    bhh2 = gru_b_hh[None, :]
    conv_b2 = conv_b[None, :]

    for _ in range(12):
        sf1 = _gather_sc(st, src1)
        sf2 = _gather_sc(st, src2)
        msg1 = _msg(hid_t1, sf1, net2_Wb, b2t)
        ag2a = _scatter_sc(msg1, dst1, zrows)
        msg2 = _msg(hid_t2, sf2, net2_Wb, b2t)
        ag2b = _scatter_sc(msg2, dst2, zrows)
        st = _dense(st, ag2a, ag2b, dg2, root_W, conv_b2, wih_t, whh_t,
                    bih2, bhh2)

    bat2 = batch.reshape(20, 500)
    bat2t = bat2.T
    out = _s2s(st, bat2, bat2t, lstm_W_ih.T, lstm_W_hh.T,
               (lstm_b_ih + lstm_b_hh)[None, :], lin3_W.T, lin3_b[None, :])
    return out


# speedup vs baseline: 1.0731x; 1.0057x over previous
import jax
import jax.numpy as jnp
from jax import lax
from jax.experimental import pallas as pl
from jax.experimental.pallas import tpu as pltpu
from jax.experimental.pallas import tpu_sc as plsc

_N = 10000
_E = 160000
_DIM = 64
_B = 512


def _leaky(v):
    return jnp.where(v >= 0, v, 0.01 * v)


def _lin0_body(x_ref, w_ref, b_ref, o_ref):
    o_ref[...] = _leaky(
        jnp.dot(x_ref[...], w_ref[...], preferred_element_type=jnp.float32)
        + b_ref[...])


def _lin0(x, w_t, b2):
    n = x.shape[0]
    return pl.pallas_call(
        _lin0_body,
        out_shape=jax.ShapeDtypeStruct((n, _DIM), jnp.float32),
    )(x, w_t, b2)


def _hidt_body(eat_ref, w_ref, b_ref, o_ref):
    o_ref[...] = _leaky(
        jnp.dot(w_ref[...], eat_ref[...], preferred_element_type=jnp.float32)
        + b_ref[...]).astype(jnp.bfloat16)


def _hidt(ea_t, net1_W, net1_bc):
    e = ea_t.shape[1]
    te = 6400
    return pl.pallas_call(
        _hidt_body,
        grid=(e // te,),
        in_specs=[
            pl.BlockSpec((4, te), lambda i: (0, i)),
            pl.BlockSpec((128, 4), lambda i: (0, 0)),
            pl.BlockSpec((128, 1), lambda i: (0, 0)),
        ],
        out_specs=pl.BlockSpec((128, te), lambda i: (0, i)),
        out_shape=jax.ShapeDtypeStruct((128, e), jnp.bfloat16),
    )(ea_t, net1_W, net1_bc)


def _msg_body(hidt_ref, sf_ref, w2_ref, b2t_ref, o_ref):
    hid = hidt_ref[...]
    sft = sf_ref[...].T
    acc = jnp.dot(b2t_ref[...], sft,
                  preferred_element_type=jnp.float32)
    for i in range(_DIM):
        part = jnp.dot(w2_ref[i * _DIM:(i + 1) * _DIM, :], hid,
                       preferred_element_type=jnp.float32)
        acc = acc + sft[i:i + 1, :] * part
    o_ref[...] = acc.T


def _msg(hid_t, sf, net2_W, b2t, te=256):
    e = sf.shape[0]
    return pl.pallas_call(
        _msg_body,
        grid=(e // te,),
        in_specs=[
            pl.BlockSpec((128, te), lambda i: (0, i)),
            pl.BlockSpec((te, _DIM), lambda i: (i, 0)),
            pl.BlockSpec((4096, 128), lambda i: (0, 0)),
            pl.BlockSpec((_DIM, _DIM), lambda i: (0, 0)),
        ],
        out_specs=pl.BlockSpec((te, _DIM), lambda i: (i, 0)),
        out_shape=jax.ShapeDtypeStruct((e, _DIM), jnp.float32),
    )(hid_t, sf, net2_W, b2t)


def _dense_body(st_ref, a0_ref, a1_ref, d0_ref, d1_ref,
                rw_ref, cb_ref, wih_ref, whh_ref, bih_ref, bhh_ref, o_ref):
    d = d0_ref[...] + d1_ref[...]
    invd = 1.0 / jnp.maximum(d, 1.0)
    aggr = (a0_ref[...] + a1_ref[...]) * invd
    st = st_ref[...]
    m = _leaky(jnp.dot(st, rw_ref[...], preferred_element_type=jnp.float32)
               + aggr + cb_ref[...])
    gi = jnp.dot(m, wih_ref[...], preferred_element_type=jnp.float32) + bih_ref[...]
    gh = jnp.dot(st, whh_ref[...], preferred_element_type=jnp.float32) + bhh_ref[...]
    r = jax.nn.sigmoid(gi[:, 0:64] + gh[:, 0:64])
    z = jax.nn.sigmoid(gi[:, 64:128] + gh[:, 64:128])
    n = jnp.tanh(gi[:, 128:192] + r * gh[:, 128:192])
    o_ref[...] = (1.0 - z) * n + z * st


def _dense(st, ag2, dg2, root_W, conv_b2, wih_t, whh_t, bih2,
           bhh2, nb=2000):
    n = st.shape[0]
    nblk = n // nb
    return pl.pallas_call(
        _dense_body,
        grid=(nblk,),
        in_specs=[
            pl.BlockSpec((nb, _DIM), lambda i: (i, 0)),
            pl.BlockSpec((nb, _DIM), lambda i: (i, 0)),
            pl.BlockSpec((nb, _DIM), lambda i, _n=nblk: (i + _n, 0)),
            pl.BlockSpec((nb, 1), lambda i: (i, 0)),
            pl.BlockSpec((nb, 1), lambda i, _n=nblk: (i + _n, 0)),
            pl.BlockSpec((_DIM, _DIM), lambda i: (0, 0)),
            pl.BlockSpec((1, _DIM), lambda i: (0, 0)),
            pl.BlockSpec((_DIM, 192), lambda i: (0, 0)),
            pl.BlockSpec((_DIM, 192), lambda i: (0, 0)),
            pl.BlockSpec((1, 192), lambda i: (0, 0)),
            pl.BlockSpec((1, 192), lambda i: (0, 0)),
        ],
        out_specs=pl.BlockSpec((nb, _DIM), lambda i: (i, 0)),
        out_shape=jax.ShapeDtypeStruct((n, _DIM), jnp.float32),
    )(st, ag2, ag2, dg2, dg2, root_W, conv_b2, wih_t, whh_t, bih2, bhh2)


def _s2s_body(out_ref, bat_ref, batt_ref, wih_ref, whh_ref, bb_ref,
              l3w_ref, l3b_ref, o_ref, e_scr):
    nblk = bat_ref.shape[0]
    nb = bat_ref.shape[1]
    grow = lax.broadcasted_iota(jnp.int32, (1, _B), 1)
    gcol = lax.broadcasted_iota(jnp.int32, (_B, 1), 0)

    q_star = jnp.zeros((_B, 2 * _DIM), jnp.float32)
    hx = jnp.zeros((_B, _DIM), jnp.float32)
    cx = jnp.zeros((_B, _DIM), jnp.float32)
    for _ in range(3):
        g = (jnp.dot(q_star, wih_ref[...], preferred_element_type=jnp.float32)
             + jnp.dot(hx, whh_ref[...], preferred_element_type=jnp.float32)
             + bb_ref[...])
        i_g = jax.nn.sigmoid(g[:, 0:64])
        f_g = jax.nn.sigmoid(g[:, 64:128])
        c_g = jnp.tanh(g[:, 128:192])
        o_g = jax.nn.sigmoid(g[:, 192:256])
        cx = f_g * cx + i_g * c_g
        hx = o_g * jnp.tanh(cx)
        q = hx

        emax_row = jnp.full((1, _B), -jnp.inf, jnp.float32)
        for b in range(nblk):
            out_blk = out_ref[b * nb:(b + 1) * nb, :]
            ids_col = batt_ref[:, b:b + 1]
            mask = ids_col == grow
            q_g = jnp.dot(mask.astype(jnp.float32), q,
                          preferred_element_type=jnp.float32)
            e_col = jnp.sum(out_blk * q_g, axis=1, keepdims=True)
            e_scr[:, b:b + 1] = e_col
            contrib = jnp.where(mask, e_col, -jnp.inf)
            emax_row = jnp.maximum(
                emax_row, jnp.max(contrib, axis=0, keepdims=True))
        emax_row = jnp.where(emax_row > -1e37, emax_row, 0.0)

        asum = jnp.zeros((_B, 1), jnp.float32)
        rread = jnp.zeros((_B, _DIM), jnp.float32)
        for b in range(nblk):
            out_blk = out_ref[b * nb:(b + 1) * nb, :]
            ids_col = batt_ref[:, b:b + 1]
            ids_row = bat_ref[b:b + 1, :]
            mask_f = (ids_col == grow).astype(jnp.float32)
            maskt_f = (gcol == ids_row).astype(jnp.float32)
            e_col = e_scr[:, b:b + 1]
            emax_g = jnp.sum(mask_f * emax_row, axis=1, keepdims=True)
            a_col = jnp.exp(e_col - emax_g)
            asum = asum + jnp.dot(maskt_f, a_col,
                                  preferred_element_type=jnp.float32)
            rread = rread + jnp.dot(maskt_f, a_col * out_blk,
                                    preferred_element_type=jnp.float32)
        rread = rread / jnp.maximum(asum, 1e-16)
        q_star = jnp.concatenate([q, rread], axis=1)

    o_ref[...] = (jnp.dot(q_star, l3w_ref[...],
                          preferred_element_type=jnp.float32) + l3b_ref[...])


def _s2s(st, bat2, bat2t, wih_t, whh_t, bb2, l3w_t, l3b2):
    return pl.pallas_call(
        _s2s_body,
        out_shape=jax.ShapeDtypeStruct((_B, 1), jnp.float32),
        scratch_shapes=[pltpu.VMEM((bat2t.shape[0], bat2.shape[0]),
                                   jnp.float32)],
    )(st, bat2, bat2t, wih_t, whh_t, bb2, l3w_t, l3b2)


def _pick_ch(e_per_w):
    best = 8
    for d in range(8, 1001, 8):
        if e_per_w % d == 0:
            best = d
    return best


def _gather_sc(table, idx):
    e = idx.shape[0]
    e_per_w = e // 32
    ch = _pick_ch(e_per_w)
    n_ch = e_per_w // ch
    mesh = plsc.VectorSubcoreMesh(core_axis_name="c", subcore_axis_name="s")

    def body(table_ref, idx_ref, out_ref, idx_v, rows_v, sem):
        wid = lax.axis_index("c") * 16 + lax.axis_index("s")
        base = wid * e_per_w

        def chunk(j, carry):
            off = base + j * ch
            pltpu.sync_copy(idx_ref.at[pl.ds(off, ch)], idx_v)
            pltpu.async_copy(table_ref.at[idx_v], rows_v, sem).wait()
            pltpu.sync_copy(rows_v, out_ref.at[pl.ds(off, ch)])
            return carry

        lax.fori_loop(0, n_ch, chunk, 0)

    f = pl.kernel(
        body,
        out_type=jax.ShapeDtypeStruct((e, _DIM), jnp.float32),
        mesh=mesh,
        scratch_types=[
            pltpu.VMEM((ch,), jnp.int32),
            pltpu.VMEM((ch, _DIM), jnp.float32),
            pltpu.SemaphoreType.DMA,
        ],
        compiler_params=pltpu.CompilerParams(use_tc_tiling_on_sc=False),
    )
    return f(table, idx)


def _scatter_sc(msg, dst, zrows):
    e = msg.shape[0]
    n = _N
    rpt = n // 16
    e_per_w = e // 32
    ch = _pick_ch(e_per_w)
    n_ch = e_per_w // ch
    mesh = plsc.VectorSubcoreMesh(core_axis_name="c", subcore_axis_name="s")

    def body(msg_ref, dst_ref, z_ref, out_ref, idx_v, buf_v, acc_sh):
        cid = lax.axis_index("c")
        sid = lax.axis_index("s")
        pltpu.sync_copy(z_ref, acc_sh.at[pl.ds(sid * rpt, rpt)])
        plsc.subcore_barrier()
        base = cid * (16 * e_per_w) + sid * e_per_w

        def chunk(j, carry):
            off = base + j * ch
            pltpu.sync_copy(dst_ref.at[pl.ds(off, ch)], idx_v)
            pltpu.sync_copy(msg_ref.at[pl.ds(off, ch)], buf_v)
            pltpu.sync_copy(buf_v, acc_sh.at[idx_v], add=True)
            return carry

        lax.fori_loop(0, n_ch, chunk, 0)
        plsc.subcore_barrier()
        pltpu.sync_copy(acc_sh.at[pl.ds(sid * rpt, rpt)],
                        out_ref.at[pl.ds(cid * n + sid * rpt, rpt)])

    f = pl.kernel(
        body,
        out_type=jax.ShapeDtypeStruct((2 * n, _DIM), jnp.float32),
        mesh=mesh,
        scratch_types=[
            pltpu.VMEM((ch,), jnp.int32),
            pltpu.VMEM((ch, _DIM), jnp.float32),
            pltpu.VMEM_SHARED((n, _DIM), jnp.float32),
        ],
        compiler_params=pltpu.CompilerParams(use_tc_tiling_on_sc=False),
    )
    return f(msg, dst, zrows)


def kernel(x, edge_index, edge_attr, batch, lin0_W, lin0_b, net1_W, net1_b,
           net2_W, net2_b, root_W, conv_b, gru_W_ih, gru_W_hh, gru_b_ih,
           gru_b_hh, lstm_W_ih, lstm_W_hh, lstm_b_ih, lstm_b_hh, lin3_W,
           lin3_b):
    src = edge_index[0]
    dst = edge_index[1]

    st = _lin0(x, lin0_W.T, lin0_b[None, :])
    hid_t = _hidt(edge_attr.T, net1_W, net1_b[:, None])
    net2_Wb = net2_W.astype(jnp.bfloat16)
    b2t = net2_b.reshape(_DIM, _DIM).T

    zrows = jnp.zeros((_N // 16, _DIM), jnp.float32)
    ones_rows = jnp.ones((_E, _DIM), jnp.float32)
    dg2 = _scatter_sc(ones_rows, dst, zrows)[:, 0:1]

    wih_t = gru_W_ih.T
    whh_t = gru_W_hh.T
    bih2 = gru_b_ih[None, :]
    bhh2 = gru_b_hh[None, :]
    conv_b2 = conv_b[None, :]

    for _ in range(12):
        sf = _gather_sc(st, src)
        msg = _msg(hid_t, sf, net2_Wb, b2t)
        ag2 = _scatter_sc(msg, dst, zrows)
        st = _dense(st, ag2, dg2, root_W, conv_b2, wih_t, whh_t,
                    bih2, bhh2)

    bat2 = batch.reshape(20, 500)
    bat2t = bat2.T
    out = _s2s(st, bat2, bat2t, lstm_W_ih.T, lstm_W_hh.T,
               (lstm_b_ih + lstm_b_hh)[None, :], lin3_W.T, lin3_b[None, :])
    return out


# speedup vs baseline: 1.1995x; 1.1178x over previous
import jax
import jax.numpy as jnp
from jax import lax
from jax.experimental import pallas as pl
from jax.experimental.pallas import tpu as pltpu
from jax.experimental.pallas import tpu_sc as plsc

_N = 10000
_E = 160000
_DIM = 64
_B = 512


def _leaky(v):
    return jnp.where(v >= 0, v, 0.01 * v)


def _lin0_body(x_ref, w_ref, b_ref, o_ref):
    o_ref[...] = _leaky(
        jnp.dot(x_ref[...], w_ref[...], preferred_element_type=jnp.float32)
        + b_ref[...])


def _lin0(x, w_t, b2):
    n = x.shape[0]
    return pl.pallas_call(
        _lin0_body,
        out_shape=jax.ShapeDtypeStruct((n, _DIM), jnp.float32),
    )(x, w_t, b2)


def _hidt_body(eat_ref, w_ref, b_ref, o_ref):
    o_ref[...] = _leaky(
        jnp.dot(w_ref[...], eat_ref[...], preferred_element_type=jnp.float32)
        + b_ref[...]).astype(jnp.bfloat16)


def _hidt(ea_t, net1_W, net1_bc):
    e = ea_t.shape[1]
    te = 6400
    return pl.pallas_call(
        _hidt_body,
        grid=(e // te,),
        in_specs=[
            pl.BlockSpec((4, te), lambda i: (0, i)),
            pl.BlockSpec((128, 4), lambda i: (0, 0)),
            pl.BlockSpec((128, 1), lambda i: (0, 0)),
        ],
        out_specs=pl.BlockSpec((128, te), lambda i: (0, i)),
        out_shape=jax.ShapeDtypeStruct((128, e), jnp.bfloat16),
    )(ea_t, net1_W, net1_bc)


def _msg_body(hidt_ref, sf_ref, w2_ref, b2t_ref, o_ref):
    te2 = sf_ref.shape[0]
    hid = hidt_ref[...]
    vt = sf_ref[...].T
    sft = jnp.concatenate([vt[0:_DIM, :], vt[_DIM:2 * _DIM, :]],
                          axis=1)
    acc = jnp.dot(b2t_ref[...], sft,
                  preferred_element_type=jnp.float32)
    for i in range(_DIM):
        part = jnp.dot(w2_ref[i * _DIM:(i + 1) * _DIM, :], hid,
                       preferred_element_type=jnp.float32)
        acc = acc + sft[i:i + 1, :] * part
    o_ref[...] = jnp.concatenate([acc[:, :te2].T, acc[:, te2:].T], axis=1)


def _msg(hid_t, sf128, net2_W, b2t, te=256):
    e2 = sf128.shape[0]
    return pl.pallas_call(
        _msg_body,
        grid=(2 * e2 // te,),
        in_specs=[
            pl.BlockSpec((128, te), lambda i: (0, i)),
            pl.BlockSpec((te // 2, 128), lambda i: (i, 0)),
            pl.BlockSpec((4096, 128), lambda i: (0, 0)),
            pl.BlockSpec((_DIM, _DIM), lambda i: (0, 0)),
        ],
        out_specs=pl.BlockSpec((te // 2, 128), lambda i: (i, 0)),
        out_shape=jax.ShapeDtypeStruct((e2, 128), jnp.float32),
    )(hid_t, sf128, net2_W, b2t)


def _dense_body(st_ref, a0_ref, a1_ref, d0_ref, d1_ref,
                rw_ref, cb_ref, wih_ref, whh_ref, bih_ref, bhh_ref, o_ref):
    d = d0_ref[...] + d1_ref[...]
    invd = 1.0 / jnp.maximum(d, 1.0)
    aggr = (a0_ref[...] + a1_ref[...]) * invd
    st = st_ref[...]
    m = _leaky(jnp.dot(st, rw_ref[...], preferred_element_type=jnp.float32)
               + aggr + cb_ref[...])
    gi = jnp.dot(m, wih_ref[...], preferred_element_type=jnp.float32) + bih_ref[...]
    gh = jnp.dot(st, whh_ref[...], preferred_element_type=jnp.float32) + bhh_ref[...]
    r = jax.nn.sigmoid(gi[:, 0:64] + gh[:, 0:64])
    z = jax.nn.sigmoid(gi[:, 64:128] + gh[:, 64:128])
    n = jnp.tanh(gi[:, 128:192] + r * gh[:, 128:192])
    o_ref[...] = (1.0 - z) * n + z * st


def _dense(st, ag2, dg2, root_W, conv_b2, wih_t, whh_t, bih2,
           bhh2, nb=2000):
    n = st.shape[0]
    nblk = n // nb
    return pl.pallas_call(
        _dense_body,
        grid=(nblk,),
        in_specs=[
            pl.BlockSpec((nb, _DIM), lambda i: (i, 0)),
            pl.BlockSpec((nb, _DIM), lambda i: (i, 0)),
            pl.BlockSpec((nb, _DIM), lambda i, _n=nblk: (i + _n, 0)),
            pl.BlockSpec((nb, 1), lambda i: (i, 0)),
            pl.BlockSpec((nb, 1), lambda i, _n=nblk: (i + _n, 0)),
            pl.BlockSpec((_DIM, _DIM), lambda i: (0, 0)),
            pl.BlockSpec((1, _DIM), lambda i: (0, 0)),
            pl.BlockSpec((_DIM, 192), lambda i: (0, 0)),
            pl.BlockSpec((_DIM, 192), lambda i: (0, 0)),
            pl.BlockSpec((1, 192), lambda i: (0, 0)),
            pl.BlockSpec((1, 192), lambda i: (0, 0)),
        ],
        out_specs=pl.BlockSpec((nb, _DIM), lambda i: (i, 0)),
        out_shape=jax.ShapeDtypeStruct((n, _DIM), jnp.float32),
    )(st, ag2, ag2, dg2, dg2, root_W, conv_b2, wih_t, whh_t, bih2, bhh2)


def _s2s_body(out_ref, bat_ref, batt_ref, wih_ref, whh_ref, bb_ref,
              l3w_ref, l3b_ref, o_ref, e_scr):
    nblk = bat_ref.shape[0]
    nb = bat_ref.shape[1]
    grow = lax.broadcasted_iota(jnp.int32, (1, _B), 1)
    gcol = lax.broadcasted_iota(jnp.int32, (_B, 1), 0)

    q_star = jnp.zeros((_B, 2 * _DIM), jnp.float32)
    hx = jnp.zeros((_B, _DIM), jnp.float32)
    cx = jnp.zeros((_B, _DIM), jnp.float32)
    for _ in range(3):
        g = (jnp.dot(q_star, wih_ref[...], preferred_element_type=jnp.float32)
             + jnp.dot(hx, whh_ref[...], preferred_element_type=jnp.float32)
             + bb_ref[...])
        i_g = jax.nn.sigmoid(g[:, 0:64])
        f_g = jax.nn.sigmoid(g[:, 64:128])
        c_g = jnp.tanh(g[:, 128:192])
        o_g = jax.nn.sigmoid(g[:, 192:256])
        cx = f_g * cx + i_g * c_g
        hx = o_g * jnp.tanh(cx)
        q = hx

        emax_row = jnp.full((1, _B), -jnp.inf, jnp.float32)
        for b in range(nblk):
            out_blk = out_ref[b * nb:(b + 1) * nb, :]
            ids_col = batt_ref[:, b:b + 1]
            mask = ids_col == grow
            q_g = jnp.dot(mask.astype(jnp.float32), q,
                          preferred_element_type=jnp.float32)
            e_col = jnp.sum(out_blk * q_g, axis=1, keepdims=True)
            e_scr[:, b:b + 1] = e_col
            contrib = jnp.where(mask, e_col, -jnp.inf)
            emax_row = jnp.maximum(
                emax_row, jnp.max(contrib, axis=0, keepdims=True))
        emax_row = jnp.where(emax_row > -1e37, emax_row, 0.0)

        asum = jnp.zeros((_B, 1), jnp.float32)
        rread = jnp.zeros((_B, _DIM), jnp.float32)
        for b in range(nblk):
            out_blk = out_ref[b * nb:(b + 1) * nb, :]
            ids_col = batt_ref[:, b:b + 1]
            ids_row = bat_ref[b:b + 1, :]
            mask_f = (ids_col == grow).astype(jnp.float32)
            maskt_f = (gcol == ids_row).astype(jnp.float32)
            e_col = e_scr[:, b:b + 1]
            emax_g = jnp.sum(mask_f * emax_row, axis=1, keepdims=True)
            a_col = jnp.exp(e_col - emax_g)
            asum = asum + jnp.dot(maskt_f, a_col,
                                  preferred_element_type=jnp.float32)
            rread = rread + jnp.dot(maskt_f, a_col * out_blk,
                                    preferred_element_type=jnp.float32)
        rread = rread / jnp.maximum(asum, 1e-16)
        q_star = jnp.concatenate([q, rread], axis=1)

    o_ref[...] = (jnp.dot(q_star, l3w_ref[...],
                          preferred_element_type=jnp.float32) + l3b_ref[...])


def _s2s(st, bat2, bat2t, wih_t, whh_t, bb2, l3w_t, l3b2):
    return pl.pallas_call(
        _s2s_body,
        out_shape=jax.ShapeDtypeStruct((_B, 1), jnp.float32),
        scratch_shapes=[pltpu.VMEM((bat2t.shape[0], bat2.shape[0]),
                                   jnp.float32)],
    )(st, bat2, bat2t, wih_t, whh_t, bb2, l3w_t, l3b2)


def _pick_ch(e_per_w):
    best = 8
    for d in range(8, 1001, 8):
        if e_per_w % d == 0:
            best = d
    return best


def _gather_sc(table, idx):
    e = idx.shape[0]
    e_per_w = e // 32
    ch = _pick_ch(e_per_w)
    n_ch = e_per_w // ch
    mesh = plsc.VectorSubcoreMesh(core_axis_name="c", subcore_axis_name="s")

    def body(table_ref, idx_ref, out_ref, idx_v, rows_v, sem):
        wid = lax.axis_index("c") * 16 + lax.axis_index("s")
        base = wid * e_per_w

        def chunk(j, carry):
            off = base + j * ch
            pltpu.sync_copy(idx_ref.at[pl.ds(off, ch)], idx_v)
            pltpu.async_copy(table_ref.at[idx_v], rows_v, sem).wait()
            pltpu.sync_copy(rows_v, out_ref.at[pl.ds(off, ch)])
            return carry

        lax.fori_loop(0, n_ch, chunk, 0)

    f = pl.kernel(
        body,
        out_type=jax.ShapeDtypeStruct((e, _DIM), jnp.float32),
        mesh=mesh,
        scratch_types=[
            pltpu.VMEM((ch,), jnp.int32),
            pltpu.VMEM((ch, _DIM), jnp.float32),
            pltpu.SemaphoreType.DMA,
        ],
        compiler_params=pltpu.CompilerParams(use_tc_tiling_on_sc=False),
    )
    return f(table, idx)


def _scatter_sc(msg, dst, zrows):
    e = msg.shape[0]
    n = _N
    rpt = n // 16
    e_per_w = e // 32
    ch = _pick_ch(e_per_w)
    n_ch = e_per_w // ch
    mesh = plsc.VectorSubcoreMesh(core_axis_name="c", subcore_axis_name="s")

    def body(msg_ref, dst_ref, z_ref, out_ref, idx_v, buf_v, acc_sh):
        cid = lax.axis_index("c")
        sid = lax.axis_index("s")
        pltpu.sync_copy(z_ref, acc_sh.at[pl.ds(sid * rpt, rpt)])
        plsc.subcore_barrier()
        base = cid * (16 * e_per_w) + sid * e_per_w

        def chunk(j, carry):
            off = base + j * ch
            pltpu.sync_copy(dst_ref.at[pl.ds(off, ch)], idx_v)
            pltpu.sync_copy(msg_ref.at[pl.ds(off, ch)], buf_v)
            pltpu.sync_copy(buf_v, acc_sh.at[idx_v], add=True)
            return carry

        lax.fori_loop(0, n_ch, chunk, 0)
        plsc.subcore_barrier()
        pltpu.sync_copy(acc_sh.at[pl.ds(sid * rpt, rpt)],
                        out_ref.at[pl.ds(cid * n + sid * rpt, rpt)])

    f = pl.kernel(
        body,
        out_type=jax.ShapeDtypeStruct((2 * n, _DIM), jnp.float32),
        mesh=mesh,
        scratch_types=[
            pltpu.VMEM((ch,), jnp.int32),
            pltpu.VMEM((ch, _DIM), jnp.float32),
            pltpu.VMEM_SHARED((n, _DIM), jnp.float32),
        ],
        compiler_params=pltpu.CompilerParams(use_tc_tiling_on_sc=False),
    )
    return f(msg, dst, zrows)


def kernel(x, edge_index, edge_attr, batch, lin0_W, lin0_b, net1_W, net1_b,
           net2_W, net2_b, root_W, conv_b, gru_W_ih, gru_W_hh, gru_b_ih,
           gru_b_hh, lstm_W_ih, lstm_W_hh, lstm_b_ih, lstm_b_hh, lin3_W,
           lin3_b):
    src = edge_index[0]
    dst = edge_index[1]
    src_p = src.reshape(_E // 256, 2, 128).transpose(0, 2, 1).reshape(_E)
    dst_p = dst.reshape(_E // 256, 2, 128).transpose(0, 2, 1).reshape(_E)

    st = _lin0(x, lin0_W.T, lin0_b[None, :])
    hid_t = _hidt(edge_attr.T, net1_W, net1_b[:, None])
    net2_Wb = net2_W.astype(jnp.bfloat16)
    b2t = net2_b.reshape(_DIM, _DIM).T

    zrows = jnp.zeros((_N // 16, _DIM), jnp.float32)
    ones_rows = jnp.ones((_E, _DIM), jnp.float32)
    dg2 = _scatter_sc(ones_rows, dst, zrows)[:, 0:1]

    wih_t = gru_W_ih.T
    whh_t = gru_W_hh.T
    bih2 = gru_b_ih[None, :]
    bhh2 = gru_b_hh[None, :]
    conv_b2 = conv_b[None, :]

    for _ in range(12):
        sf = _gather_sc(st, src_p)
        sf128 = sf.reshape(_E // 2, 128)
        msg128 = _msg(hid_t, sf128, net2_Wb, b2t)
        msg = msg128.reshape(_E, _DIM)
        ag2 = _scatter_sc(msg, dst_p, zrows)
        st = _dense(st, ag2, dg2, root_W, conv_b2, wih_t, whh_t,
                    bih2, bhh2)

    bat2 = batch.reshape(20, 500)
    bat2t = bat2.T
    out = _s2s(st, bat2, bat2t, lstm_W_ih.T, lstm_W_hh.T,
               (lstm_b_ih + lstm_b_hh)[None, :], lin3_W.T, lin3_b[None, :])
    return out


# speedup vs baseline: 1.2686x; 1.0576x over previous
import jax
import jax.numpy as jnp
from jax import lax
from jax.experimental import pallas as pl
from jax.experimental.pallas import tpu as pltpu
from jax.experimental.pallas import tpu_sc as plsc

_N = 10000
_E = 160000
_DIM = 64
_B = 512


def _leaky(v):
    return jnp.where(v >= 0, v, 0.01 * v)


def _lin0_body(x_ref, w_ref, b_ref, o_ref):
    o_ref[...] = _leaky(
        jnp.dot(x_ref[...], w_ref[...], preferred_element_type=jnp.float32)
        + b_ref[...])


def _lin0(x, w_t, b2):
    n = x.shape[0]
    return pl.pallas_call(
        _lin0_body,
        out_shape=jax.ShapeDtypeStruct((n, _DIM), jnp.float32),
    )(x, w_t, b2)


def _hidt_body(eat_ref, w_ref, b_ref, o_ref):
    o_ref[...] = _leaky(
        jnp.dot(w_ref[...], eat_ref[...], preferred_element_type=jnp.float32)
        + b_ref[...]).astype(jnp.bfloat16)


def _hidt(ea_t, net1_W, net1_bc):
    e = ea_t.shape[1]
    te = 6400
    return pl.pallas_call(
        _hidt_body,
        grid=(e // te,),
        in_specs=[
            pl.BlockSpec((4, te), lambda i: (0, i)),
            pl.BlockSpec((128, 4), lambda i: (0, 0)),
            pl.BlockSpec((128, 1), lambda i: (0, 0)),
        ],
        out_specs=pl.BlockSpec((128, te), lambda i: (0, i)),
        out_shape=jax.ShapeDtypeStruct((128, e), jnp.bfloat16),
    )(ea_t, net1_W, net1_bc)


def _msg_body(hidt_ref, sf_ref, w2_ref, b2t_ref, o_ref):
    te2 = sf_ref.shape[0]
    hid = hidt_ref[...]
    vt = sf_ref[...].T
    sft = jnp.concatenate([vt[0:_DIM, :], vt[_DIM:2 * _DIM, :]],
                          axis=1)
    acc = jnp.dot(b2t_ref[...], sft,
                  preferred_element_type=jnp.float32)
    for i in range(_DIM):
        part = jnp.dot(w2_ref[i * _DIM:(i + 1) * _DIM, :], hid,
                       preferred_element_type=jnp.float32)
        acc = acc + sft[i:i + 1, :] * part
    o_ref[...] = jnp.concatenate([acc[:, :te2].T, acc[:, te2:].T], axis=1)


def _msg(hid_t, sf128, net2_W, b2t, te=640):
    e2 = sf128.shape[0]
    return pl.pallas_call(
        _msg_body,
        grid=(2 * e2 // te,),
        in_specs=[
            pl.BlockSpec((128, te), lambda i: (0, i)),
            pl.BlockSpec((te // 2, 128), lambda i: (i, 0)),
            pl.BlockSpec((4096, 128), lambda i: (0, 0)),
            pl.BlockSpec((_DIM, _DIM), lambda i: (0, 0)),
        ],
        out_specs=pl.BlockSpec((te // 2, 128), lambda i: (i, 0)),
        out_shape=jax.ShapeDtypeStruct((e2, 128), jnp.float32),
    )(hid_t, sf128, net2_W, b2t)


def _dense_body(st_ref, a0_ref, a1_ref, d0_ref, d1_ref,
                rw_ref, cb_ref, wih_ref, whh_ref, bih_ref, bhh_ref, o_ref):
    d = d0_ref[...] + d1_ref[...]
    invd = 1.0 / jnp.maximum(d, 1.0)
    aggr = (a0_ref[...] + a1_ref[...]) * invd
    st = st_ref[...]
    m = _leaky(jnp.dot(st, rw_ref[...], preferred_element_type=jnp.float32)
               + aggr + cb_ref[...])
    gi = jnp.dot(m, wih_ref[...], preferred_element_type=jnp.float32) + bih_ref[...]
    gh = jnp.dot(st, whh_ref[...], preferred_element_type=jnp.float32) + bhh_ref[...]
    r = jax.nn.sigmoid(gi[:, 0:64] + gh[:, 0:64])
    z = jax.nn.sigmoid(gi[:, 64:128] + gh[:, 64:128])
    n = jnp.tanh(gi[:, 128:192] + r * gh[:, 128:192])
    o_ref[...] = (1.0 - z) * n + z * st


def _dense(st, ag2, dg2, root_W, conv_b2, wih_t, whh_t, bih2,
           bhh2, nb=2000):
    n = st.shape[0]
    nblk = n // nb
    return pl.pallas_call(
        _dense_body,
        grid=(nblk,),
        in_specs=[
            pl.BlockSpec((nb, _DIM), lambda i: (i, 0)),
            pl.BlockSpec((nb, _DIM), lambda i: (i, 0)),
            pl.BlockSpec((nb, _DIM), lambda i, _n=nblk: (i + _n, 0)),
            pl.BlockSpec((nb, 1), lambda i: (i, 0)),
            pl.BlockSpec((nb, 1), lambda i, _n=nblk: (i + _n, 0)),
            pl.BlockSpec((_DIM, _DIM), lambda i: (0, 0)),
            pl.BlockSpec((1, _DIM), lambda i: (0, 0)),
            pl.BlockSpec((_DIM, 192), lambda i: (0, 0)),
            pl.BlockSpec((_DIM, 192), lambda i: (0, 0)),
            pl.BlockSpec((1, 192), lambda i: (0, 0)),
            pl.BlockSpec((1, 192), lambda i: (0, 0)),
        ],
        out_specs=pl.BlockSpec((nb, _DIM), lambda i: (i, 0)),
        out_shape=jax.ShapeDtypeStruct((n, _DIM), jnp.float32),
    )(st, ag2, ag2, dg2, dg2, root_W, conv_b2, wih_t, whh_t, bih2, bhh2)


def _s2s_body(out_ref, bat_ref, batt_ref, wih_ref, whh_ref, bb_ref,
              l3w_ref, l3b_ref, o_ref, e_scr):
    nblk = bat_ref.shape[0]
    nb = bat_ref.shape[1]
    grow = lax.broadcasted_iota(jnp.int32, (1, _B), 1)
    gcol = lax.broadcasted_iota(jnp.int32, (_B, 1), 0)

    q_star = jnp.zeros((_B, 2 * _DIM), jnp.float32)
    hx = jnp.zeros((_B, _DIM), jnp.float32)
    cx = jnp.zeros((_B, _DIM), jnp.float32)
    for _ in range(3):
        g = (jnp.dot(q_star, wih_ref[...], preferred_element_type=jnp.float32)
             + jnp.dot(hx, whh_ref[...], preferred_element_type=jnp.float32)
             + bb_ref[...])
        i_g = jax.nn.sigmoid(g[:, 0:64])
        f_g = jax.nn.sigmoid(g[:, 64:128])
        c_g = jnp.tanh(g[:, 128:192])
        o_g = jax.nn.sigmoid(g[:, 192:256])
        cx = f_g * cx + i_g * c_g
        hx = o_g * jnp.tanh(cx)
        q = hx

        emax_row = jnp.full((1, _B), -jnp.inf, jnp.float32)
        for b in range(nblk):
            out_blk = out_ref[b * nb:(b + 1) * nb, :]
            ids_col = batt_ref[:, b:b + 1]
            mask = ids_col == grow
            q_g = jnp.dot(mask.astype(jnp.float32), q,
                          preferred_element_type=jnp.float32)
            e_col = jnp.sum(out_blk * q_g, axis=1, keepdims=True)
            e_scr[:, b:b + 1] = e_col
            contrib = jnp.where(mask, e_col, -jnp.inf)
            emax_row = jnp.maximum(
                emax_row, jnp.max(contrib, axis=0, keepdims=True))
        emax_row = jnp.where(emax_row > -1e37, emax_row, 0.0)

        asum = jnp.zeros((_B, 1), jnp.float32)
        rread = jnp.zeros((_B, _DIM), jnp.float32)
        for b in range(nblk):
            out_blk = out_ref[b * nb:(b + 1) * nb, :]
            ids_col = batt_ref[:, b:b + 1]
            ids_row = bat_ref[b:b + 1, :]
            mask_f = (ids_col == grow).astype(jnp.float32)
            maskt_f = (gcol == ids_row).astype(jnp.float32)
            e_col = e_scr[:, b:b + 1]
            emax_g = jnp.sum(mask_f * emax_row, axis=1, keepdims=True)
            a_col = jnp.exp(e_col - emax_g)
            asum = asum + jnp.dot(maskt_f, a_col,
                                  preferred_element_type=jnp.float32)
            rread = rread + jnp.dot(maskt_f, a_col * out_blk,
                                    preferred_element_type=jnp.float32)
        rread = rread / jnp.maximum(asum, 1e-16)
        q_star = jnp.concatenate([q, rread], axis=1)

    o_ref[...] = (jnp.dot(q_star, l3w_ref[...],
                          preferred_element_type=jnp.float32) + l3b_ref[...])


def _s2s(st, bat2, bat2t, wih_t, whh_t, bb2, l3w_t, l3b2):
    return pl.pallas_call(
        _s2s_body,
        out_shape=jax.ShapeDtypeStruct((_B, 1), jnp.float32),
        scratch_shapes=[pltpu.VMEM((bat2t.shape[0], bat2.shape[0]),
                                   jnp.float32)],
    )(st, bat2, bat2t, wih_t, whh_t, bb2, l3w_t, l3b2)


def _pick_ch(e_per_w):
    best = 8
    for d in range(8, 1001, 8):
        if e_per_w % d == 0:
            best = d
    return best


def _gather_sc(table, idx):
    e = idx.shape[0]
    e_per_w = e // 32
    ch = _pick_ch(e_per_w)
    n_ch = e_per_w // ch
    mesh = plsc.VectorSubcoreMesh(core_axis_name="c", subcore_axis_name="s")

    def body(table_ref, idx_ref, out_ref, idx_v, rows_v, sem):
        wid = lax.axis_index("c") * 16 + lax.axis_index("s")
        base = wid * e_per_w

        def chunk(j, carry):
            off = base + j * ch
            pltpu.sync_copy(idx_ref.at[pl.ds(off, ch)], idx_v)
            pltpu.async_copy(table_ref.at[idx_v], rows_v, sem).wait()
            pltpu.sync_copy(rows_v, out_ref.at[pl.ds(off, ch)])
            return carry

        lax.fori_loop(0, n_ch, chunk, 0)

    f = pl.kernel(
        body,
        out_type=jax.ShapeDtypeStruct((e, _DIM), jnp.float32),
        mesh=mesh,
        scratch_types=[
            pltpu.VMEM((ch,), jnp.int32),
            pltpu.VMEM((ch, _DIM), jnp.float32),
            pltpu.SemaphoreType.DMA,
        ],
        compiler_params=pltpu.CompilerParams(use_tc_tiling_on_sc=False),
    )
    return f(table, idx)


def _scatter_sc(msg, dst, zrows):
    e = msg.shape[0]
    n = _N
    rpt = n // 16
    e_per_w = e // 32
    ch = _pick_ch(e_per_w)
    n_ch = e_per_w // ch
    mesh = plsc.VectorSubcoreMesh(core_axis_name="c", subcore_axis_name="s")

    def body(msg_ref, dst_ref, z_ref, out_ref, idx_v, buf_v, acc_sh):
        cid = lax.axis_index("c")
        sid = lax.axis_index("s")
        pltpu.sync_copy(z_ref, acc_sh.at[pl.ds(sid * rpt, rpt)])
        plsc.subcore_barrier()
        base = cid * (16 * e_per_w) + sid * e_per_w

        def chunk(j, carry):
            off = base + j * ch
            pltpu.sync_copy(dst_ref.at[pl.ds(off, ch)], idx_v)
            pltpu.sync_copy(msg_ref.at[pl.ds(off, ch)], buf_v)
            pltpu.sync_copy(buf_v, acc_sh.at[idx_v], add=True)
            return carry

        lax.fori_loop(0, n_ch, chunk, 0)
        plsc.subcore_barrier()
        pltpu.sync_copy(acc_sh.at[pl.ds(sid * rpt, rpt)],
                        out_ref.at[pl.ds(cid * n + sid * rpt, rpt)])

    f = pl.kernel(
        body,
        out_type=jax.ShapeDtypeStruct((2 * n, _DIM), jnp.float32),
        mesh=mesh,
        scratch_types=[
            pltpu.VMEM((ch,), jnp.int32),
            pltpu.VMEM((ch, _DIM), jnp.float32),
            pltpu.VMEM_SHARED((n, _DIM), jnp.float32),
        ],
        compiler_params=pltpu.CompilerParams(use_tc_tiling_on_sc=False),
    )
    return f(msg, dst, zrows)


def kernel(x, edge_index, edge_attr, batch, lin0_W, lin0_b, net1_W, net1_b,
           net2_W, net2_b, root_W, conv_b, gru_W_ih, gru_W_hh, gru_b_ih,
           gru_b_hh, lstm_W_ih, lstm_W_hh, lstm_b_ih, lstm_b_hh, lin3_W,
           lin3_b):
    src = edge_index[0]
    dst = edge_index[1]
    src_p = src.reshape(_E // 640, 2, 320).transpose(0, 2, 1).reshape(_E)
    dst_p = dst.reshape(_E // 640, 2, 320).transpose(0, 2, 1).reshape(_E)

    st = _lin0(x, lin0_W.T, lin0_b[None, :])
    hid_t = _hidt(edge_attr.T, net1_W, net1_b[:, None])
    net2_Wb = net2_W.astype(jnp.bfloat16)
    b2t = net2_b.reshape(_DIM, _DIM).T

    zrows = jnp.zeros((_N // 16, _DIM), jnp.float32)
    ones_rows = jnp.ones((_E, _DIM), jnp.float32)
    dg2 = _scatter_sc(ones_rows, dst, zrows)[:, 0:1]

    wih_t = gru_W_ih.T
    whh_t = gru_W_hh.T
    bih2 = gru_b_ih[None, :]
    bhh2 = gru_b_hh[None, :]
    conv_b2 = conv_b[None, :]

    for _ in range(12):
        sf = _gather_sc(st, src_p)
        sf128 = sf.reshape(_E // 2, 128)
        msg128 = _msg(hid_t, sf128, net2_Wb, b2t)
        msg = msg128.reshape(_E, _DIM)
        ag2 = _scatter_sc(msg, dst_p, zrows)
        st = _dense(st, ag2, dg2, root_W, conv_b2, wih_t, whh_t,
                    bih2, bhh2)

    bat2 = batch.reshape(20, 500)
    bat2t = bat2.T
    out = _s2s(st, bat2, bat2t, lstm_W_ih.T, lstm_W_hh.T,
               (lstm_b_ih + lstm_b_hh)[None, :], lin3_W.T, lin3_b[None, :])
    return out
